# baseline (speedup 1.0000x reference)
import functools

import jax
import jax.numpy as jnp
from jax import lax
from jax.experimental import pallas as pl
from jax.experimental.pallas import tpu as pltpu
from jax.experimental.pallas import tpu_sc as plsc

_NC, _NS, _LANES = 2, 16, 16
_NW = _NC * _NS


def _wid():
    return lax.axis_index("s") * _NC + lax.axis_index("c")


def _sc_mesh():
    return plsc.VectorSubcoreMesh(
        core_axis_name="c", subcore_axis_name="s",
        num_cores=_NC, num_subcores=_NS)


@functools.cache
def _cls_gather(V, D, B):
    bpw = B // _NW

    @functools.partial(
        pl.kernel, mesh=_sc_mesh(),
        out_type=jax.ShapeDtypeStruct((B, D), jnp.float32),
        scratch_types=[
            pltpu.VMEM((bpw,), jnp.int32),
            pltpu.VMEM((bpw, D), jnp.float32),
            pltpu.SemaphoreType.DMA,
        ])
    def k(table_hbm, idx_hbm, out_hbm, idx_v, rows_v, sem):
        base = _wid() * bpw
        pltpu.sync_copy(idx_hbm.at[pl.ds(base, bpw)], idx_v)
        pltpu.async_copy(table_hbm.at[idx_v], rows_v, sem).wait()
        pltpu.sync_copy(rows_v, out_hbm.at[pl.ds(base, bpw)])

    return k


@functools.cache
def _encoder(B, D, BM=256):
    def body(emb_ref, w_ref, b_ref, out_ref):
        out_ref[...] = jnp.tanh(
            jnp.dot(emb_ref[...], w_ref[...],
                    preferred_element_type=jnp.float32) + b_ref[...])

    return pl.pallas_call(
        body,
        grid=(B // BM,),
        in_specs=[
            pl.BlockSpec((BM, D), lambda i: (i, 0)),
            pl.BlockSpec((D, D), lambda i: (0, 0)),
            pl.BlockSpec((1, D), lambda i: (0, 0)),
        ],
        out_specs=pl.BlockSpec((BM, D), lambda i: (i, 0)),
        out_shape=jax.ShapeDtypeStruct((B, D), jnp.float32),
    )


@functools.cache
def _score_partial(DN, D, B, K):
    bpw = B // _NW
    nj = D // _LANES

    @functools.partial(
        pl.kernel, mesh=_sc_mesh(),
        out_type=jax.ShapeDtypeStruct((B, K, _LANES), jnp.float32),
        scratch_types=[
            pltpu.VMEM((bpw, K), jnp.int32),
            pltpu.VMEM((bpw, D), jnp.float32),
            pltpu.VMEM((K, D), jnp.float32),
            pltpu.VMEM((K, _LANES), jnp.float32),
            pltpu.SemaphoreType.DMA,
        ])
    def k(dict_hbm, idx_hbm, q_hbm, out_hbm, idx_v, q_v, rows_v, part_v, sem):
        base = _wid() * bpw
        pltpu.sync_copy(idx_hbm.at[pl.ds(base, bpw)], idx_v)
        pltpu.sync_copy(q_hbm.at[pl.ds(base, bpw)], q_v)

        KT = 16

        def b_body(b, carry):
            pltpu.async_copy(dict_hbm.at[idx_v.at[b]], rows_v, sem).wait()

            def kt_body(kt, c2):
                kb = kt * KT
                q0 = q_v[b, pl.ds(0, _LANES)]
                accs = [rows_v[kb + i, pl.ds(0, _LANES)] * q0
                        for i in range(KT)]
                for j in range(1, nj):
                    qj = q_v[b, pl.ds(_LANES * j, _LANES)]
                    for i in range(KT):
                        accs[i] = accs[i] + (
                            rows_v[kb + i, pl.ds(_LANES * j, _LANES)] * qj)
                for i in range(KT):
                    part_v[kb + i, pl.ds(0, _LANES)] = accs[i]
                return c2

            lax.fori_loop(0, K // KT, kt_body, 0, unroll=False)
            pltpu.sync_copy(part_v, out_hbm.at[base + b])
            return carry

        lax.fori_loop(0, bpw, b_body, 0, unroll=False)

    return k


@functools.cache
def _finalize(B, K, BM=256):
    KL = K * _LANES

    def body(p_ref, out_ref):
        fold = jnp.equal(
            lax.broadcasted_iota(jnp.int32, (KL, K), 0) // _LANES,
            lax.broadcasted_iota(jnp.int32, (KL, K), 1)).astype(jnp.float32)
        out_ref[...] = jnp.dot(p_ref[...], fold,
                               preferred_element_type=jnp.float32)

    return pl.pallas_call(
        body,
        grid=(B // BM,),
        in_specs=[pl.BlockSpec((BM, KL), lambda i: (i, 0))],
        out_specs=pl.BlockSpec((BM, K), lambda i: (i, 0)),
        out_shape=jax.ShapeDtypeStruct((B, K), jnp.float32),
    )


def kernel(input_ids, attention_mask, topk_cand_idxs, word_emb, W_enc, b_enc,
           dict_embs):
    B, _ = input_ids.shape
    K = topk_cand_idxs.shape[1]
    V, D = word_emb.shape
    DN = dict_embs.shape[0]

    idx0 = input_ids[:, 0].astype(jnp.int32)
    emb0 = _cls_gather(V, D, B)(word_emb, idx0)
    q = _encoder(B, D)(emb0, W_enc, b_enc.reshape(1, D))

    cidx = jnp.clip(topk_cand_idxs, 0, DN - 1).astype(jnp.int32)
    part = _score_partial(DN, D, B, K)(dict_embs, cidx, q)
    score = _finalize(B, K)(part.reshape(B, K * _LANES))

    mask0 = attention_mask[:, 0].astype(jnp.float32)
    valid = (topk_cand_idxs >= 0).astype(jnp.float32)
    return score * mask0[:, None] * valid


# speedup vs baseline: 2.1090x; 2.1090x over previous
import functools

import jax
import jax.numpy as jnp
from jax import lax
from jax.experimental import pallas as pl
from jax.experimental.pallas import tpu as pltpu
from jax.experimental.pallas import tpu_sc as plsc

_NC, _NS, _LANES = 2, 16, 16
_NW = _NC * _NS


def _wid():
    return lax.axis_index("s") * _NC + lax.axis_index("c")


def _sc_mesh():
    return plsc.VectorSubcoreMesh(
        core_axis_name="c", subcore_axis_name="s",
        num_cores=_NC, num_subcores=_NS)


@functools.cache
def _cls_gather(V, D, B):
    bpw = B // _NW

    @functools.partial(
        pl.kernel, mesh=_sc_mesh(),
        out_type=jax.ShapeDtypeStruct((B, D), jnp.float32),
        scratch_types=[
            pltpu.VMEM((bpw,), jnp.int32),
            pltpu.VMEM((bpw, D), jnp.float32),
            pltpu.SemaphoreType.DMA,
        ])
    def k(table_hbm, idx_hbm, out_hbm, idx_v, rows_v, sem):
        base = _wid() * bpw
        pltpu.sync_copy(idx_hbm.at[pl.ds(base, bpw)], idx_v)
        pltpu.async_copy(table_hbm.at[idx_v], rows_v, sem).wait()
        pltpu.sync_copy(rows_v, out_hbm.at[pl.ds(base, bpw)])

    return k


@functools.cache
def _encoder(B, D, BM=256):
    def body(emb_ref, w_ref, b_ref, out_ref):
        out_ref[...] = jnp.tanh(
            jnp.dot(emb_ref[...], w_ref[...],
                    preferred_element_type=jnp.float32) + b_ref[...])

    return pl.pallas_call(
        body,
        grid=(B // BM,),
        in_specs=[
            pl.BlockSpec((BM, D), lambda i: (i, 0)),
            pl.BlockSpec((D, D), lambda i: (0, 0)),
            pl.BlockSpec((1, D), lambda i: (0, 0)),
        ],
        out_specs=pl.BlockSpec((BM, D), lambda i: (i, 0)),
        out_shape=jax.ShapeDtypeStruct((B, D), jnp.float32),
    )


@functools.cache
def _score_partial(DN, D, B, K):
    bpw = B // _NW
    nj = D // _LANES

    K2 = K // 2

    @functools.partial(
            pl.kernel, mesh=_sc_mesh(),
            out_type=jax.ShapeDtypeStruct((B, K, _LANES), jnp.float32),
            scratch_types=[
                pltpu.VMEM((bpw, K), jnp.int32),
                pltpu.VMEM((bpw, D), jnp.float32),
                pltpu.VMEM((K2, D), jnp.float32),
                pltpu.VMEM((K2, D), jnp.float32),
                pltpu.VMEM((K, _LANES), jnp.float32),
                pltpu.SemaphoreType.DMA,
                pltpu.SemaphoreType.DMA,
            ])
    def k(dict_hbm, idx_hbm, q_hbm, out_hbm, idx_v, q_v, rows0_v, rows1_v,
          part_v, sem0, sem1):
        base = _wid() * bpw
        pltpu.sync_copy(idx_hbm.at[pl.ds(base, bpw)], idx_v)
        pltpu.sync_copy(q_hbm.at[pl.ds(base, bpw)], q_v)

        KT = 8

        def compute(b, half, rows_v):
            def kt_body(kt, c2):
                kb = kt * KT
                q0 = q_v[b, pl.ds(0, _LANES)]
                accs = [rows_v[kb + i, pl.ds(0, _LANES)] * q0
                        for i in range(KT)]
                for j in range(1, nj):
                    qj = q_v[b, pl.ds(_LANES * j, _LANES)]
                    for i in range(KT):
                        accs[i] = accs[i] + (
                            rows_v[kb + i, pl.ds(_LANES * j, _LANES)]
                            * qj)
                for i in range(KT):
                    part_v[half * K2 + kb + i, pl.ds(0, _LANES)] = accs[i]
                return c2

            lax.fori_loop(0, K2 // KT, kt_body, 0, unroll=False)

        pltpu.async_copy(
            dict_hbm.at[idx_v.at[0, pl.ds(0, K2)]], rows0_v, sem0)

        def b_body(b, carry):
            pltpu.async_copy(
                dict_hbm.at[idx_v.at[b, pl.ds(K2, K2)]], rows1_v, sem1)
            pltpu.make_async_copy(
                dict_hbm.at[idx_v.at[b, pl.ds(0, K2)]], rows0_v,
                sem0).wait()
            compute(b, 0, rows0_v)
            nb = lax.rem(b + 1, bpw)
            pltpu.async_copy(
                dict_hbm.at[idx_v.at[nb, pl.ds(0, K2)]], rows0_v, sem0)
            pltpu.make_async_copy(
                dict_hbm.at[idx_v.at[b, pl.ds(K2, K2)]], rows1_v,
                sem1).wait()
            compute(b, 1, rows1_v)
            pltpu.sync_copy(part_v, out_hbm.at[base + b])
            return carry

        lax.fori_loop(0, bpw, b_body, 0, unroll=False)
        pltpu.make_async_copy(
            dict_hbm.at[idx_v.at[0, pl.ds(0, K2)]], rows0_v, sem0).wait()

    return k


@functools.cache
def _finalize(B, K, BM=256):
    KL = K * _LANES

    def body(p_ref, out_ref):
        fold = jnp.equal(
            lax.broadcasted_iota(jnp.int32, (KL, K), 0) // _LANES,
            lax.broadcasted_iota(jnp.int32, (KL, K), 1)).astype(jnp.float32)
        out_ref[...] = jnp.dot(p_ref[...], fold,
                               preferred_element_type=jnp.float32)

    return pl.pallas_call(
        body,
        grid=(B // BM,),
        in_specs=[pl.BlockSpec((BM, KL), lambda i: (i, 0))],
        out_specs=pl.BlockSpec((BM, K), lambda i: (i, 0)),
        out_shape=jax.ShapeDtypeStruct((B, K), jnp.float32),
    )


def kernel(input_ids, attention_mask, topk_cand_idxs, word_emb, W_enc, b_enc,
           dict_embs):
    B, _ = input_ids.shape
    K = topk_cand_idxs.shape[1]
    V, D = word_emb.shape
    DN = dict_embs.shape[0]

    idx0 = input_ids[:, 0].astype(jnp.int32)
    emb0 = _cls_gather(V, D, B)(word_emb, idx0)
    q = _encoder(B, D)(emb0, W_enc, b_enc.reshape(1, D))

    cidx = jnp.clip(topk_cand_idxs, 0, DN - 1).astype(jnp.int32)
    part = _score_partial(DN, D, B, K)(dict_embs, cidx, q)
    score = _finalize(B, K)(part.reshape(B, K * _LANES))

    mask0 = attention_mask[:, 0].astype(jnp.float32)
    valid = (topk_cand_idxs >= 0).astype(jnp.float32)
    return score * mask0[:, None] * valid


# speedup vs baseline: 2.1249x; 1.0076x over previous
import functools

import jax
import jax.numpy as jnp
from jax import lax
from jax.experimental import pallas as pl
from jax.experimental.pallas import tpu as pltpu
from jax.experimental.pallas import tpu_sc as plsc

_NC, _NS, _LANES = 2, 16, 16
_NW = _NC * _NS


def _wid():
    return lax.axis_index("s") * _NC + lax.axis_index("c")


def _sc_mesh():
    return plsc.VectorSubcoreMesh(
        core_axis_name="c", subcore_axis_name="s",
        num_cores=_NC, num_subcores=_NS)


@functools.cache
def _cls_gather(V, D, B):
    bpw = B // _NW

    @functools.partial(
        pl.kernel, mesh=_sc_mesh(),
        out_type=jax.ShapeDtypeStruct((B, D), jnp.float32),
        scratch_types=[
            pltpu.VMEM((bpw,), jnp.int32),
            pltpu.VMEM((bpw, D), jnp.float32),
            pltpu.SemaphoreType.DMA,
        ])
    def k(table_hbm, idx_hbm, out_hbm, idx_v, rows_v, sem):
        base = _wid() * bpw
        pltpu.sync_copy(idx_hbm.at[pl.ds(base, bpw)], idx_v)
        pltpu.async_copy(table_hbm.at[idx_v], rows_v, sem).wait()
        pltpu.sync_copy(rows_v, out_hbm.at[pl.ds(base, bpw)])

    return k


@functools.cache
def _encoder(B, D, BM=256):
    def body(emb_ref, w_ref, b_ref, out_ref):
        out_ref[...] = jnp.tanh(
            jnp.dot(emb_ref[...], w_ref[...],
                    preferred_element_type=jnp.float32) + b_ref[...])

    return pl.pallas_call(
        body,
        grid=(B // BM,),
        in_specs=[
            pl.BlockSpec((BM, D), lambda i: (i, 0)),
            pl.BlockSpec((D, D), lambda i: (0, 0)),
            pl.BlockSpec((1, D), lambda i: (0, 0)),
        ],
        out_specs=pl.BlockSpec((BM, D), lambda i: (i, 0)),
        out_shape=jax.ShapeDtypeStruct((B, D), jnp.float32),
    )


@functools.cache
def _score_partial(DN, D, B, K):
    bpw = B // _NW
    nj = D // _LANES

    K2 = K // 2

    @functools.partial(
            pl.kernel, mesh=_sc_mesh(),
            out_type=jax.ShapeDtypeStruct((B, K, _LANES), jnp.float32),
            scratch_types=[
                pltpu.VMEM((bpw, K), jnp.int32),
                pltpu.VMEM((bpw, D), jnp.float32),
                pltpu.VMEM((K2, D), jnp.float32),
                pltpu.VMEM((K2, D), jnp.float32),
                pltpu.VMEM((K, _LANES), jnp.float32),
                pltpu.SemaphoreType.DMA,
                pltpu.SemaphoreType.DMA,
            ])
    def k(dict_hbm, idx_hbm, q_hbm, out_hbm, idx_v, q_v, rows0_v, rows1_v,
          part_v, sem0, sem1):
        base = _wid() * bpw
        pltpu.sync_copy(idx_hbm.at[pl.ds(base, bpw)], idx_v)
        pltpu.sync_copy(q_hbm.at[pl.ds(base, bpw)], q_v)

        KT = 4

        def compute(b, half, rows_v):
            def kt_body(kt, c2):
                kb = kt * KT
                q0 = q_v[b, pl.ds(0, _LANES)]
                accs = [rows_v[kb + i, pl.ds(0, _LANES)] * q0
                        for i in range(KT)]
                for j in range(1, nj):
                    qj = q_v[b, pl.ds(_LANES * j, _LANES)]
                    for i in range(KT):
                        accs[i] = accs[i] + (
                            rows_v[kb + i, pl.ds(_LANES * j, _LANES)]
                            * qj)
                for i in range(KT):
                    part_v[half * K2 + kb + i, pl.ds(0, _LANES)] = accs[i]
                return c2

            lax.fori_loop(0, K2 // KT, kt_body, 0, unroll=False)

        pltpu.async_copy(
            dict_hbm.at[idx_v.at[0, pl.ds(0, K2)]], rows0_v, sem0)

        def b_body(b, carry):
            pltpu.async_copy(
                dict_hbm.at[idx_v.at[b, pl.ds(K2, K2)]], rows1_v, sem1)
            pltpu.make_async_copy(
                dict_hbm.at[idx_v.at[b, pl.ds(0, K2)]], rows0_v,
                sem0).wait()
            compute(b, 0, rows0_v)
            nb = lax.rem(b + 1, bpw)
            pltpu.async_copy(
                dict_hbm.at[idx_v.at[nb, pl.ds(0, K2)]], rows0_v, sem0)
            pltpu.make_async_copy(
                dict_hbm.at[idx_v.at[b, pl.ds(K2, K2)]], rows1_v,
                sem1).wait()
            compute(b, 1, rows1_v)
            pltpu.sync_copy(part_v, out_hbm.at[base + b])
            return carry

        lax.fori_loop(0, bpw, b_body, 0, unroll=False)
        pltpu.make_async_copy(
            dict_hbm.at[idx_v.at[0, pl.ds(0, K2)]], rows0_v, sem0).wait()

    return k


@functools.cache
def _finalize(B, K, BM=256):
    KL = K * _LANES

    def body(p_ref, out_ref):
        fold = jnp.equal(
            lax.broadcasted_iota(jnp.int32, (KL, K), 0) // _LANES,
            lax.broadcasted_iota(jnp.int32, (KL, K), 1)).astype(jnp.float32)
        out_ref[...] = jnp.dot(p_ref[...], fold,
                               preferred_element_type=jnp.float32)

    return pl.pallas_call(
        body,
        grid=(B // BM,),
        in_specs=[pl.BlockSpec((BM, KL), lambda i: (i, 0))],
        out_specs=pl.BlockSpec((BM, K), lambda i: (i, 0)),
        out_shape=jax.ShapeDtypeStruct((B, K), jnp.float32),
    )


def kernel(input_ids, attention_mask, topk_cand_idxs, word_emb, W_enc, b_enc,
           dict_embs):
    B, _ = input_ids.shape
    K = topk_cand_idxs.shape[1]
    V, D = word_emb.shape
    DN = dict_embs.shape[0]

    idx0 = input_ids[:, 0].astype(jnp.int32)
    emb0 = _cls_gather(V, D, B)(word_emb, idx0)
    q = _encoder(B, D)(emb0, W_enc, b_enc.reshape(1, D))

    cidx = jnp.clip(topk_cand_idxs, 0, DN - 1).astype(jnp.int32)
    part = _score_partial(DN, D, B, K)(dict_embs, cidx, q)
    score = _finalize(B, K)(part.reshape(B, K * _LANES))

    mask0 = attention_mask[:, 0].astype(jnp.float32)
    valid = (topk_cand_idxs >= 0).astype(jnp.float32)
    return score * mask0[:, None] * valid


# speedup vs baseline: 2.2189x; 1.0442x over previous
import functools

import jax
import jax.numpy as jnp
from jax import lax
from jax.experimental import pallas as pl
from jax.experimental.pallas import tpu as pltpu
from jax.experimental.pallas import tpu_sc as plsc

_NC, _NS, _LANES = 2, 16, 16
_NW = _NC * _NS


def _wid():
    return lax.axis_index("s") * _NC + lax.axis_index("c")


def _sc_mesh():
    return plsc.VectorSubcoreMesh(
        core_axis_name="c", subcore_axis_name="s",
        num_cores=_NC, num_subcores=_NS)


@functools.cache
def _cls_gather(V, D, B):
    bpw = B // _NW

    @functools.partial(
        pl.kernel, mesh=_sc_mesh(),
        out_type=jax.ShapeDtypeStruct((B, D), jnp.float32),
        scratch_types=[
            pltpu.VMEM((bpw,), jnp.int32),
            pltpu.VMEM((bpw, D), jnp.float32),
            pltpu.SemaphoreType.DMA,
        ])
    def k(table_hbm, idx_hbm, out_hbm, idx_v, rows_v, sem):
        base = _wid() * bpw
        pltpu.sync_copy(idx_hbm.at[pl.ds(base, bpw)], idx_v)
        pltpu.async_copy(table_hbm.at[idx_v], rows_v, sem).wait()
        pltpu.sync_copy(rows_v, out_hbm.at[pl.ds(base, bpw)])

    return k


@functools.cache
def _encoder(B, D, BM=256):
    def body(emb_ref, w_ref, b_ref, out_ref):
        out_ref[...] = jnp.tanh(
            jnp.dot(emb_ref[...], w_ref[...],
                    preferred_element_type=jnp.float32) + b_ref[...])

    return pl.pallas_call(
        body,
        grid=(B // BM,),
        in_specs=[
            pl.BlockSpec((BM, D), lambda i: (i, 0)),
            pl.BlockSpec((D, D), lambda i: (0, 0)),
            pl.BlockSpec((1, D), lambda i: (0, 0)),
        ],
        out_specs=pl.BlockSpec((BM, D), lambda i: (i, 0)),
        out_shape=jax.ShapeDtypeStruct((B, D), jnp.float32),
    )


@functools.cache
def _score_partial(DN, D, B, K):
    bpw = B // _NW
    nj = D // _LANES

    NB = 4
    KC = K // NB

    @functools.partial(
            pl.kernel, mesh=_sc_mesh(),
            out_type=jax.ShapeDtypeStruct((B, K, _LANES), jnp.float32),
            scratch_types=[
                pltpu.VMEM((bpw, K), jnp.int32),
                pltpu.VMEM((bpw, D), jnp.float32),
                pltpu.VMEM((KC, D), jnp.float32),
                pltpu.VMEM((KC, D), jnp.float32),
                pltpu.VMEM((KC, D), jnp.float32),
                pltpu.VMEM((KC, D), jnp.float32),
                pltpu.VMEM((K, _LANES), jnp.float32),
                pltpu.SemaphoreType.DMA,
                pltpu.SemaphoreType.DMA,
                pltpu.SemaphoreType.DMA,
                pltpu.SemaphoreType.DMA,
            ])
    def k(dict_hbm, idx_hbm, q_hbm, out_hbm, idx_v, q_v, rb0, rb1, rb2, rb3,
          part_v, s0, s1, s2, s3):
        bufs = (rb0, rb1, rb2, rb3)
        sems = (s0, s1, s2, s3)
        base = _wid() * bpw
        pltpu.sync_copy(idx_hbm.at[pl.ds(base, bpw)], idx_v)
        pltpu.sync_copy(q_hbm.at[pl.ds(base, bpw)], q_v)

        KT = 4

        def issue(b, c):
            pltpu.async_copy(
                dict_hbm.at[idx_v.at[b, pl.ds(c * KC, KC)]], bufs[c], sems[c])

        def drain(b, c):
            pltpu.make_async_copy(
                dict_hbm.at[idx_v.at[b, pl.ds(c * KC, KC)]], bufs[c],
                sems[c]).wait()

        def compute(b, c):
            rows_v = bufs[c]

            def kt_body(kt, c2):
                kb = kt * KT
                q0 = q_v[b, pl.ds(0, _LANES)]
                accs = [rows_v[kb + i, pl.ds(0, _LANES)] * q0
                        for i in range(KT)]
                for j in range(1, nj):
                    qj = q_v[b, pl.ds(_LANES * j, _LANES)]
                    for i in range(KT):
                        accs[i] = accs[i] + (
                            rows_v[kb + i, pl.ds(_LANES * j, _LANES)]
                            * qj)
                for i in range(KT):
                    part_v[c * KC + kb + i, pl.ds(0, _LANES)] = accs[i]
                return c2

            lax.fori_loop(0, KC // KT, kt_body, 0, unroll=False)

        for c in range(NB - 1):
            issue(0, c)

        def b_body(b, carry):
            nb2 = lax.rem(b + 1, bpw)
            issue(b, NB - 1)
            for c in range(NB - 1):
                drain(b, c)
                compute(b, c)
                issue(nb2, c)
            drain(b, NB - 1)
            compute(b, NB - 1)
            pltpu.sync_copy(part_v, out_hbm.at[base + b])
            return carry

        lax.fori_loop(0, bpw, b_body, 0, unroll=False)
        for c in range(NB - 1):
            drain(0, c)

    return k


@functools.cache
def _finalize(B, K, BM=256):
    KL = K * _LANES

    def body(p_ref, out_ref):
        fold = jnp.equal(
            lax.broadcasted_iota(jnp.int32, (KL, K), 0) // _LANES,
            lax.broadcasted_iota(jnp.int32, (KL, K), 1)).astype(jnp.float32)
        out_ref[...] = jnp.dot(p_ref[...], fold,
                               preferred_element_type=jnp.float32)

    return pl.pallas_call(
        body,
        grid=(B // BM,),
        in_specs=[pl.BlockSpec((BM, KL), lambda i: (i, 0))],
        out_specs=pl.BlockSpec((BM, K), lambda i: (i, 0)),
        out_shape=jax.ShapeDtypeStruct((B, K), jnp.float32),
    )


def kernel(input_ids, attention_mask, topk_cand_idxs, word_emb, W_enc, b_enc,
           dict_embs):
    B, _ = input_ids.shape
    K = topk_cand_idxs.shape[1]
    V, D = word_emb.shape
    DN = dict_embs.shape[0]

    idx0 = input_ids[:, 0].astype(jnp.int32)
    emb0 = _cls_gather(V, D, B)(word_emb, idx0)
    q = _encoder(B, D)(emb0, W_enc, b_enc.reshape(1, D))

    cidx = jnp.clip(topk_cand_idxs, 0, DN - 1).astype(jnp.int32)
    part = _score_partial(DN, D, B, K)(dict_embs, cidx, q)
    score = _finalize(B, K)(part.reshape(B, K * _LANES))

    mask0 = attention_mask[:, 0].astype(jnp.float32)
    valid = (topk_cand_idxs >= 0).astype(jnp.float32)
    return score * mask0[:, None] * valid
